# Initial kernel scaffold; baseline (speedup 1.0000x reference)
import jax
import jax.numpy as jnp
from jax.experimental import pallas as pl


def kernel(x, edge_index, Wl1, bl1, Wr1, g1, be1, Wl2, bl2, Wr2, g2, be2, Wl3, bl3, Wr3, Wf1, bf1, Wf2, bf2):
    raise NotImplementedError("write your pallas kernel here")


# speedup vs baseline: 3.2069x; 3.2069x over previous
import functools

import jax
import jax.numpy as jnp
from jax import lax
from jax.experimental import pallas as pl
from jax.experimental.pallas import tpu as pltpu
from jax.experimental.pallas import tpu_sc as plsc

N = 10000
E = 320000
NC = 2
NS = 16
NW = NC * NS
CHUNK = 128
EPW = 10240
NCH = EPW // CHUNK
NPAD = 10240
ROWS_PER_TILE = NPAD // NS
WB = ROWS_PER_TILE // CHUNK
DEGW = 16


NCH2 = NCH // 2


def _make_sc_agg(width):
    mesh = plsc.VectorSubcoreMesh(core_axis_name="c", subcore_axis_name="s")
    scratch = [
        pltpu.VMEM((NCH2, CHUNK), jnp.int32),
        pltpu.VMEM((NCH2, CHUNK), jnp.int32),
        pltpu.VMEM((CHUNK, width), jnp.float32),
        pltpu.VMEM_SHARED((NPAD, width), jnp.float32),
        pltpu.SemaphoreType.DMA,
    ]

    @functools.partial(
        pl.kernel, mesh=mesh,
        out_type=jax.ShapeDtypeStruct((NC, NPAD, width), jnp.float32),
        scratch_types=scratch)
    def k(table, srcb, dstb, zrow, out, src_v, dst_v, buf, acc, sem):
        c = lax.axis_index("c")
        s = lax.axis_index("s")
        wid = s * NC + c
        r0 = s * ROWS_PER_TILE

        pltpu.sync_copy(zrow, buf)
        for i in range(WB):
            pltpu.sync_copy(buf, acc.at[pl.ds(r0 + i * CHUNK, CHUNK)])
        plsc.subcore_barrier()

        def body(i, carry):
            pltpu.async_copy(table.at[src_v.at[i]], buf, sem).wait()
            pltpu.sync_copy(buf, acc.at[dst_v.at[i]], add=True)
            return carry

        for ph in range(2):
            pltpu.sync_copy(srcb.at[wid, pl.ds(ph * NCH2, NCH2)], src_v)
            pltpu.sync_copy(dstb.at[wid, pl.ds(ph * NCH2, NCH2)], dst_v)
            lax.fori_loop(0, NCH2, body, 0)
        plsc.subcore_barrier()

        for i in range(WB):
            pltpu.sync_copy(acc.at[pl.ds(r0 + i * CHUNK, CHUNK)], buf)
            pltpu.sync_copy(buf, out.at[c, pl.ds(r0 + i * CHUNK, CHUNK)])

    return k


def _make_sc_deg():
    mesh = plsc.VectorSubcoreMesh(core_axis_name="c", subcore_axis_name="s")
    scratch = [
        pltpu.VMEM((NCH, CHUNK), jnp.int32),
        pltpu.VMEM((CHUNK, DEGW), jnp.float32),
        pltpu.VMEM_SHARED((NPAD, DEGW), jnp.float32),
    ]

    @functools.partial(
        pl.kernel, mesh=mesh,
        out_type=jax.ShapeDtypeStruct((NC, NPAD, DEGW), jnp.float32),
        scratch_types=scratch)
    def k(dstb, orow, dout, dst_v, ones_v, dacc):
        c = lax.axis_index("c")
        s = lax.axis_index("s")
        wid = s * NC + c
        r0 = s * ROWS_PER_TILE

        pltpu.sync_copy(orow.at[1], ones_v)
        for i in range(WB):
            pltpu.sync_copy(ones_v, dacc.at[pl.ds(r0 + i * CHUNK, CHUNK)])
        pltpu.sync_copy(orow.at[0], ones_v)
        pltpu.sync_copy(dstb.at[wid], dst_v)
        plsc.subcore_barrier()

        def body(i, carry):
            pltpu.sync_copy(ones_v, dacc.at[dst_v.at[i]], add=True)
            return carry

        lax.fori_loop(0, NCH, body, 0)
        plsc.subcore_barrier()

        for i in range(WB):
            pltpu.sync_copy(dacc.at[pl.ds(r0 + i * CHUNK, CHUNK)], ones_v)
            pltpu.sync_copy(ones_v, dout.at[c, pl.ds(r0 + i * CHUNK, CHUNK)])

    return k


def _proj2(x, wl, wr):
    def body(x_ref, wl_ref, wr_ref, p_ref, r_ref):
        xv = x_ref[...]
        p_ref[...] = jnp.dot(xv, wl_ref[...], preferred_element_type=jnp.float32)
        r_ref[...] = jnp.dot(xv, wr_ref[...], preferred_element_type=jnp.float32)

    return pl.pallas_call(
        body,
        out_shape=(
            jax.ShapeDtypeStruct((N, wl.shape[1]), jnp.float32),
            jax.ShapeDtypeStruct((N, wr.shape[1]), jnp.float32),
        ),
    )(x, wl, wr)


def _combine_bn_proj(sp, degp, r, bl, g, be, wln, wrn):
    def body(s_ref, d_ref, r_ref, bl_ref, g_ref, be_ref, wl_ref, wr_ref,
             p_ref, rn_ref):
        sv = s_ref[0, :N, :] + s_ref[1, :N, :]
        deg = d_ref[0, :N, :1] + d_ref[1, :N, :1]
        inv = 1.0 / jnp.maximum(deg, 1.0)
        t = sv * inv + bl_ref[...] + r_ref[...]
        m = jnp.mean(t, axis=0, keepdims=True)
        v = jnp.mean((t - m) * (t - m), axis=0, keepdims=True)
        h = (t - m) * lax.rsqrt(v + 1e-5) * g_ref[...] + be_ref[...]
        h = jnp.maximum(h, 0.0)
        p_ref[...] = jnp.dot(h, wl_ref[...], preferred_element_type=jnp.float32)
        rn_ref[...] = jnp.dot(h, wr_ref[...], preferred_element_type=jnp.float32)

    return pl.pallas_call(
        body,
        out_shape=(
            jax.ShapeDtypeStruct((N, wln.shape[1]), jnp.float32),
            jax.ShapeDtypeStruct((N, wrn.shape[1]), jnp.float32),
        ),
    )(sp, degp, r, bl.reshape(1, -1), g.reshape(1, -1), be.reshape(1, -1),
      wln, wrn)


def _combine_bn_keep(sp, degp, r, bl, g, be, wrn):
    def body(s_ref, d_ref, r_ref, bl_ref, g_ref, be_ref, wr_ref,
             h_ref, rn_ref):
        sv = s_ref[0, :N, :] + s_ref[1, :N, :]
        deg = d_ref[0, :N, :1] + d_ref[1, :N, :1]
        inv = 1.0 / jnp.maximum(deg, 1.0)
        t = sv * inv + bl_ref[...] + r_ref[...]
        m = jnp.mean(t, axis=0, keepdims=True)
        v = jnp.mean((t - m) * (t - m), axis=0, keepdims=True)
        h = (t - m) * lax.rsqrt(v + 1e-5) * g_ref[...] + be_ref[...]
        h = jnp.maximum(h, 0.0)
        h_ref[...] = h
        rn_ref[...] = jnp.dot(h, wr_ref[...], preferred_element_type=jnp.float32)

    return pl.pallas_call(
        body,
        out_shape=(
            jax.ShapeDtypeStruct((N, sp.shape[2]), jnp.float32),
            jax.ShapeDtypeStruct((N, wrn.shape[1]), jnp.float32),
        ),
    )(sp, degp, r, bl.reshape(1, -1), g.reshape(1, -1), be.reshape(1, -1),
      wrn)


def _head(sp, degp, r, wl3, bl3, wf1, bf1, wf2, bf2):
    def body(s_ref, d_ref, r_ref, wl_ref, bl_ref, w1_ref, b1_ref, w2_ref,
             b2_ref, o_ref):
        sv = s_ref[0, :N, :] + s_ref[1, :N, :]
        deg = d_ref[0, :N, :1] + d_ref[1, :N, :1]
        inv = 1.0 / jnp.maximum(deg, 1.0)
        agg = jnp.dot(sv * inv, wl_ref[...], preferred_element_type=jnp.float32)
        h = jnp.maximum(agg + bl_ref[...] + r_ref[...], 0.0)
        z = jnp.dot(h, w1_ref[...], preferred_element_type=jnp.float32)
        z = jnp.maximum(z + b1_ref[...], 0.0)
        z = jnp.dot(z, w2_ref[...], preferred_element_type=jnp.float32)
        z = z + b2_ref[...]
        zm = z - jnp.max(z, axis=1, keepdims=True)
        lse = jnp.log(jnp.sum(jnp.exp(zm), axis=1, keepdims=True))
        o_ref[...] = zm - lse

    return pl.pallas_call(
        body,
        out_shape=jax.ShapeDtypeStruct((N, wf2.shape[1]), jnp.float32),
    )(sp, degp, r, wl3, bl3.reshape(1, -1), wf1, bf1.reshape(1, -1), wf2,
      bf2.reshape(1, -1))


def kernel(x, edge_index, Wl1, bl1, Wr1, g1, be1, Wl2, bl2, Wr2, g2, be2,
           Wl3, bl3, Wr3, Wf1, bf1, Wf2, bf2):
    pad = NW * EPW - E
    src = jnp.concatenate(
        [edge_index[0], jnp.zeros((pad,), jnp.int32)]).reshape(NW, NCH, CHUNK)
    dst = jnp.concatenate(
        [edge_index[1], jnp.full((pad,), N, jnp.int32)]).reshape(NW, NCH, CHUNK)
    z128 = jnp.zeros((CHUNK, 128), jnp.float32)
    oz = jnp.stack([jnp.ones((CHUNK, DEGW), jnp.float32),
                    jnp.zeros((CHUNK, DEGW), jnp.float32)])

    agg128 = _make_sc_agg(128)
    degk = _make_sc_deg()

    degp = degk(dst, oz)
    p1, r1 = _proj2(x, Wl1, Wr1)
    s1p = agg128(p1, src, dst, z128)
    p2, r2 = _combine_bn_proj(s1p, degp, r1, bl1, g1, be1, Wl2, Wr2)
    s2p = agg128(p2, src, dst, z128)
    h2, r3 = _combine_bn_keep(s2p, degp, r2, bl2, g2, be2, Wr3)
    s3p = agg128(h2, src, dst, z128)
    return _head(s3p, degp, r3, Wl3, bl3, Wf1, bf1, Wf2, bf2)


# speedup vs baseline: 3.5734x; 1.1143x over previous
import functools

import jax
import jax.numpy as jnp
from jax import lax
from jax.experimental import pallas as pl
from jax.experimental.pallas import tpu as pltpu
from jax.experimental.pallas import tpu_sc as plsc

N = 10000
E = 320000
NC = 2
NS = 16
NW = NC * NS
CHUNK = 128
EPW = 10240
NCH = EPW // CHUNK
NPAD = 10240
ROWS_PER_TILE = NPAD // NS
WB = ROWS_PER_TILE // CHUNK
DEGW = 16


NCH2 = NCH // 2


def _make_sc_agg(width):
    mesh = plsc.VectorSubcoreMesh(core_axis_name="c", subcore_axis_name="s")
    scratch = [
        pltpu.VMEM((NCH2, CHUNK), jnp.int32),
        pltpu.VMEM((NCH2, CHUNK), jnp.int32),
        pltpu.VMEM((CHUNK, width), jnp.float32),
        pltpu.VMEM((CHUNK, width), jnp.float32),
        pltpu.VMEM_SHARED((NPAD, width), jnp.float32),
        pltpu.SemaphoreType.DMA,
        pltpu.SemaphoreType.DMA,
    ]
    PAIRS = NCH2 // 2

    @functools.partial(
        pl.kernel, mesh=mesh,
        out_type=jax.ShapeDtypeStruct((NC, NPAD, width), jnp.float32),
        scratch_types=scratch)
    def k(table, srcb, dstb, zrow, out, src_v, dst_v, bufa, bufb, acc,
          sema, semb):
        c = lax.axis_index("c")
        s = lax.axis_index("s")
        wid = s * NC + c
        r0 = s * ROWS_PER_TILE

        pltpu.sync_copy(zrow, bufa)
        for i in range(WB):
            pltpu.sync_copy(bufa, acc.at[pl.ds(r0 + i * CHUNK, CHUNK)])
        plsc.subcore_barrier()

        def body(g, carry):
            pltpu.make_async_copy(table.at[src_v.at[0]], bufa, sema).wait()
            pltpu.sync_copy(bufa, acc.at[dst_v.at[2 * g]], add=True)

            @pl.when(g < PAIRS - 1)
            def _():
                pltpu.async_copy(table.at[src_v.at[2 * g + 2]], bufa, sema)

            pltpu.make_async_copy(table.at[src_v.at[0]], bufb, semb).wait()
            pltpu.sync_copy(bufb, acc.at[dst_v.at[2 * g + 1]], add=True)

            @pl.when(g < PAIRS - 1)
            def _():
                pltpu.async_copy(table.at[src_v.at[2 * g + 3]], bufb, semb)

            return carry

        for ph in range(2):
            pltpu.sync_copy(srcb.at[wid, pl.ds(ph * NCH2, NCH2)], src_v)
            pltpu.sync_copy(dstb.at[wid, pl.ds(ph * NCH2, NCH2)], dst_v)
            pltpu.async_copy(table.at[src_v.at[0]], bufa, sema)
            pltpu.async_copy(table.at[src_v.at[1]], bufb, semb)
            lax.fori_loop(0, PAIRS, body, 0)
        plsc.subcore_barrier()

        for i in range(WB):
            pltpu.sync_copy(acc.at[pl.ds(r0 + i * CHUNK, CHUNK)], bufa)
            pltpu.sync_copy(bufa, out.at[c, pl.ds(r0 + i * CHUNK, CHUNK)])

    return k


def _make_sc_deg():
    mesh = plsc.VectorSubcoreMesh(core_axis_name="c", subcore_axis_name="s")
    scratch = [
        pltpu.VMEM((NCH, CHUNK), jnp.int32),
        pltpu.VMEM((CHUNK, DEGW), jnp.float32),
        pltpu.VMEM_SHARED((NPAD, DEGW), jnp.float32),
    ]

    @functools.partial(
        pl.kernel, mesh=mesh,
        out_type=jax.ShapeDtypeStruct((NC, NPAD, DEGW), jnp.float32),
        scratch_types=scratch)
    def k(dstb, orow, dout, dst_v, ones_v, dacc):
        c = lax.axis_index("c")
        s = lax.axis_index("s")
        wid = s * NC + c
        r0 = s * ROWS_PER_TILE

        pltpu.sync_copy(orow.at[1], ones_v)
        for i in range(WB):
            pltpu.sync_copy(ones_v, dacc.at[pl.ds(r0 + i * CHUNK, CHUNK)])
        pltpu.sync_copy(orow.at[0], ones_v)
        pltpu.sync_copy(dstb.at[wid], dst_v)
        plsc.subcore_barrier()

        def body(i, carry):
            pltpu.sync_copy(ones_v, dacc.at[dst_v.at[i]], add=True)
            return carry

        lax.fori_loop(0, NCH, body, 0)
        plsc.subcore_barrier()

        for i in range(WB):
            pltpu.sync_copy(dacc.at[pl.ds(r0 + i * CHUNK, CHUNK)], ones_v)
            pltpu.sync_copy(ones_v, dout.at[c, pl.ds(r0 + i * CHUNK, CHUNK)])

    return k


def _proj2(x, wl, wr):
    def body(x_ref, wl_ref, wr_ref, p_ref, r_ref):
        xv = x_ref[...]
        p_ref[...] = jnp.dot(xv, wl_ref[...], preferred_element_type=jnp.float32)
        r_ref[...] = jnp.dot(xv, wr_ref[...], preferred_element_type=jnp.float32)

    return pl.pallas_call(
        body,
        out_shape=(
            jax.ShapeDtypeStruct((N, wl.shape[1]), jnp.float32),
            jax.ShapeDtypeStruct((N, wr.shape[1]), jnp.float32),
        ),
    )(x, wl, wr)


def _combine_bn_proj(sp, degp, r, bl, g, be, wln, wrn):
    def body(s_ref, d_ref, r_ref, bl_ref, g_ref, be_ref, wl_ref, wr_ref,
             p_ref, rn_ref):
        sv = s_ref[0, :N, :] + s_ref[1, :N, :]
        deg = d_ref[0, :N, :1] + d_ref[1, :N, :1]
        inv = 1.0 / jnp.maximum(deg, 1.0)
        t = sv * inv + bl_ref[...] + r_ref[...]
        m = jnp.mean(t, axis=0, keepdims=True)
        v = jnp.mean((t - m) * (t - m), axis=0, keepdims=True)
        h = (t - m) * lax.rsqrt(v + 1e-5) * g_ref[...] + be_ref[...]
        h = jnp.maximum(h, 0.0)
        p_ref[...] = jnp.dot(h, wl_ref[...], preferred_element_type=jnp.float32)
        rn_ref[...] = jnp.dot(h, wr_ref[...], preferred_element_type=jnp.float32)

    return pl.pallas_call(
        body,
        out_shape=(
            jax.ShapeDtypeStruct((N, wln.shape[1]), jnp.float32),
            jax.ShapeDtypeStruct((N, wrn.shape[1]), jnp.float32),
        ),
    )(sp, degp, r, bl.reshape(1, -1), g.reshape(1, -1), be.reshape(1, -1),
      wln, wrn)


def _combine_bn_keep(sp, degp, r, bl, g, be, wrn):
    def body(s_ref, d_ref, r_ref, bl_ref, g_ref, be_ref, wr_ref,
             h_ref, rn_ref):
        sv = s_ref[0, :N, :] + s_ref[1, :N, :]
        deg = d_ref[0, :N, :1] + d_ref[1, :N, :1]
        inv = 1.0 / jnp.maximum(deg, 1.0)
        t = sv * inv + bl_ref[...] + r_ref[...]
        m = jnp.mean(t, axis=0, keepdims=True)
        v = jnp.mean((t - m) * (t - m), axis=0, keepdims=True)
        h = (t - m) * lax.rsqrt(v + 1e-5) * g_ref[...] + be_ref[...]
        h = jnp.maximum(h, 0.0)
        h_ref[...] = h
        rn_ref[...] = jnp.dot(h, wr_ref[...], preferred_element_type=jnp.float32)

    return pl.pallas_call(
        body,
        out_shape=(
            jax.ShapeDtypeStruct((N, sp.shape[2]), jnp.float32),
            jax.ShapeDtypeStruct((N, wrn.shape[1]), jnp.float32),
        ),
    )(sp, degp, r, bl.reshape(1, -1), g.reshape(1, -1), be.reshape(1, -1),
      wrn)


def _head(sp, degp, r, wl3, bl3, wf1, bf1, wf2, bf2):
    def body(s_ref, d_ref, r_ref, wl_ref, bl_ref, w1_ref, b1_ref, w2_ref,
             b2_ref, o_ref):
        sv = s_ref[0, :N, :] + s_ref[1, :N, :]
        deg = d_ref[0, :N, :1] + d_ref[1, :N, :1]
        inv = 1.0 / jnp.maximum(deg, 1.0)
        agg = jnp.dot(sv * inv, wl_ref[...], preferred_element_type=jnp.float32)
        h = jnp.maximum(agg + bl_ref[...] + r_ref[...], 0.0)
        z = jnp.dot(h, w1_ref[...], preferred_element_type=jnp.float32)
        z = jnp.maximum(z + b1_ref[...], 0.0)
        z = jnp.dot(z, w2_ref[...], preferred_element_type=jnp.float32)
        z = z + b2_ref[...]
        zm = z - jnp.max(z, axis=1, keepdims=True)
        lse = jnp.log(jnp.sum(jnp.exp(zm), axis=1, keepdims=True))
        o_ref[...] = zm - lse

    return pl.pallas_call(
        body,
        out_shape=jax.ShapeDtypeStruct((N, wf2.shape[1]), jnp.float32),
    )(sp, degp, r, wl3, bl3.reshape(1, -1), wf1, bf1.reshape(1, -1), wf2,
      bf2.reshape(1, -1))


def kernel(x, edge_index, Wl1, bl1, Wr1, g1, be1, Wl2, bl2, Wr2, g2, be2,
           Wl3, bl3, Wr3, Wf1, bf1, Wf2, bf2):
    pad = NW * EPW - E
    src = jnp.concatenate(
        [edge_index[0], jnp.zeros((pad,), jnp.int32)]).reshape(NW, NCH, CHUNK)
    dst = jnp.concatenate(
        [edge_index[1], jnp.full((pad,), N, jnp.int32)]).reshape(NW, NCH, CHUNK)
    z128 = jnp.zeros((CHUNK, 128), jnp.float32)
    oz = jnp.stack([jnp.ones((CHUNK, DEGW), jnp.float32),
                    jnp.zeros((CHUNK, DEGW), jnp.float32)])

    agg128 = _make_sc_agg(128)
    degk = _make_sc_deg()

    degp = degk(dst, oz)
    p1, r1 = _proj2(x, Wl1, Wr1)
    s1p = agg128(p1, src, dst, z128)
    p2, r2 = _combine_bn_proj(s1p, degp, r1, bl1, g1, be1, Wl2, Wr2)
    s2p = agg128(p2, src, dst, z128)
    h2, r3 = _combine_bn_keep(s2p, degp, r2, bl2, g2, be2, Wr3)
    s3p = agg128(h2, src, dst, z128)
    return _head(s3p, degp, r3, Wl3, bl3, Wf1, bf1, Wf2, bf2)
